# baseline (speedup 1.0000x reference)
import functools

import numpy as np
import jax
import jax.numpy as jnp
from jax import lax
from jax.experimental import pallas as pl
from jax.experimental.pallas import tpu as pltpu
from jax.experimental.pallas import tpu_sc as plsc

_PRIMES = [49999, 49993, 49991, 49957, 49943, 49939, 49937, 49927]
_H = len(_PRIMES)
_HC = 4
_HID = 1024
_DH = 128
_KSZ = 4
_DIL = 3
_EPS_G = float(np.finfo(np.float32).eps)
_EPS_C = 1e-5
_OFFSETS = np.concatenate(
    [[0], np.cumsum(np.asarray(_PRIMES[:-1], dtype=np.int64))]
).astype(np.int32)

_NW = 32
_CHUNK = 128


def _sc_gather(table, idx):
    nw, nchunks, c = idx.shape
    rows_total = nw * nchunks * c
    per_worker = nchunks * c
    mesh = plsc.VectorSubcoreMesh(core_axis_name="c", subcore_axis_name="s")

    @functools.partial(
        pl.kernel,
        out_type=jax.ShapeDtypeStruct((rows_total, _DH), jnp.float32),
        mesh=mesh,
        scratch_types=[
            pltpu.VMEM((nchunks, c), jnp.int32),
            pltpu.VMEM((c, _DH), jnp.float32),
            pltpu.VMEM((c, _DH), jnp.float32),
            pltpu.SemaphoreType.DMA,
            pltpu.SemaphoreType.DMA,
        ],
    )
    def gather_kernel(table_hbm, idx_hbm, out_hbm, idx_v, buf0, buf1, sem0, sem1):
        wid = lax.axis_index("s") * 2 + lax.axis_index("c")
        base = wid * per_worker
        pltpu.sync_copy(idx_hbm.at[wid], idx_v)

        bufs = (buf0, buf1)
        sems = (sem0, sem1)
        pltpu.async_copy(table_hbm.at[idx_v.at[0]], buf0, sem0)

        def step(j, _):
            def even_odd(parity):
                buf, sem = bufs[parity], sems[parity]
                nbuf, nsem = bufs[1 - parity], sems[1 - parity]

                @pl.when(j + 1 < nchunks)
                def _():
                    pltpu.async_copy(table_hbm.at[idx_v.at[j + 1]], nbuf, nsem)

                pltpu.make_async_copy(table_hbm.at[idx_v.at[j]], buf, sem).wait()
                pltpu.sync_copy(buf, out_hbm.at[pl.ds(base + j * c, c)])

            @pl.when(j % 2 == 0)
            def _():
                even_odd(0)

            @pl.when(j % 2 == 1)
            def _():
                even_odd(1)

            return 0

        lax.fori_loop(0, nchunks, step, 0)

    return gather_kernel(table, idx)


def _dense(emb, hid, wcat, g_k, g_h, norms_w, convw):
    n = emb.shape[1]
    t = 256
    grid = n // t
    cpb = 2048 // t
    halo = 16

    def body(emb_ref, hid_ref, w_ref, gk_ref, gh_ref, nw_ref, cw_ref,
             out_ref, tail_ref):
        i = pl.program_id(0)
        seq_start = (i % cpb) == 0
        eb = emb_ref[...].astype(jnp.bfloat16)
        p = jnp.zeros((t, (1 + _HC) * _HID), jnp.float32)
        for hh in range(_H):
            p = p + lax.dot_general(eb[hh], w_ref[:, _DH * hh:_DH * (hh + 1)],
                                    (((1,), (1,)), ((), ())),
                                    preferred_element_type=jnp.float32)
        value = p[:, :_HID]
        for m in range(_HC):
            k = p[:, _HID * (m + 1):_HID * (m + 2)]
            nk = k * lax.rsqrt(jnp.mean(k * k, axis=-1, keepdims=True)
                               + _EPS_G) * gk_ref[m][None, :]
            q = hid_ref[:, _HID * m:_HID * (m + 1)]
            nq = q * lax.rsqrt(jnp.mean(q * q, axis=-1, keepdims=True)
                               + _EPS_G) * gh_ref[m][None, :]
            g = jnp.sum(nk * nq, axis=-1, keepdims=True) / np.sqrt(float(_HID))
            g = jnp.sqrt(jnp.clip(jnp.abs(g), 1e-6, None)) * jnp.sign(g)
            gate = jax.nn.sigmoid(g)
            gated = gate * value
            xs = gated * lax.rsqrt(jnp.mean(gated * gated, axis=-1,
                                            keepdims=True) + _EPS_C) \
                * nw_ref[m][None, :]
            prev_tail = jnp.where(seq_start, 0.0, tail_ref[m])
            xfull = jnp.concatenate([prev_tail, xs], axis=0)
            co = jnp.zeros_like(xs)
            for kk in range(_KSZ):
                off = halo - (_KSZ - 1) * _DIL + _DIL * kk
                co = co + xfull[off:off + t, :] \
                    * cw_ref[kk, _HID * m:_HID * (m + 1)][None, :]
            co = co * jax.nn.sigmoid(co)
            tail_ref[m] = xs[t - halo:, :]
            out_ref[:, _HID * m:_HID * (m + 1)] = q + co + gated

    return pl.pallas_call(
        body,
        grid=(grid,),
        in_specs=[
            pl.BlockSpec((_H, t, _DH), lambda i: (0, i, 0)),
            pl.BlockSpec((t, _HC * _HID), lambda i: (i, 0)),
            pl.BlockSpec(((1 + _HC) * _HID, _H * _DH), lambda i: (0, 0)),
            pl.BlockSpec((_HC, _HID), lambda i: (0, 0)),
            pl.BlockSpec((_HC, _HID), lambda i: (0, 0)),
            pl.BlockSpec((_HC, _HID), lambda i: (0, 0)),
            pl.BlockSpec((_KSZ, _HC * _HID), lambda i: (0, 0)),
        ],
        out_specs=pl.BlockSpec((t, _HC * _HID), lambda i: (i, 0)),
        out_shape=jax.ShapeDtypeStruct((n, _HC * _HID), jnp.float32),
        scratch_shapes=[pltpu.VMEM((_HC, 16, _HID), jnp.float32)],
    )(emb, hid, wcat, g_k, g_h, norms_w, convw)


def kernel(hash_indices, hidden_states, table, w_v, w_k, g_k, g_h,
           norms_w, conv_w):
    b, s, h = hash_indices.shape
    n = b * s
    idx = (hash_indices.astype(jnp.int32).reshape(n, _H).T
           + jnp.asarray(_OFFSETS, jnp.int32)[:, None])
    idx = idx.reshape(_NW, -1, _CHUNK)
    emb = _sc_gather(table, idx).reshape(_H, n, _DH)
    hid = hidden_states.reshape(n, _HC * _HID)
    wcat = jnp.concatenate(
        [w_v, w_k.reshape(_HC * _HID, _H * _DH)], axis=0
    ).astype(jnp.bfloat16)
    convw = conv_w.reshape(_HC * _HID, _KSZ).T
    out = _dense(emb, hid, wcat, g_k, g_h, norms_w, convw)
    return out.reshape(b, s, _HC, _HID)


# speedup vs baseline: 1.0599x; 1.0599x over previous
import functools

import numpy as np
import jax
import jax.numpy as jnp
from jax import lax
from jax.experimental import pallas as pl
from jax.experimental.pallas import tpu as pltpu
from jax.experimental.pallas import tpu_sc as plsc

_PRIMES = [49999, 49993, 49991, 49957, 49943, 49939, 49937, 49927]
_H = len(_PRIMES)
_HC = 4
_HID = 1024
_DH = 128
_KSZ = 4
_DIL = 3
_EPS_G = float(np.finfo(np.float32).eps)
_EPS_C = 1e-5
_OFFSETS = np.concatenate(
    [[0], np.cumsum(np.asarray(_PRIMES[:-1], dtype=np.int64))]
).astype(np.int32)

_NW = 32
_CHUNK = 128


def _sc_gather(table, idx):
    nw, nchunks, c = idx.shape
    rows_total = nw * nchunks * c
    per_worker = nchunks * c
    mesh = plsc.VectorSubcoreMesh(core_axis_name="c", subcore_axis_name="s")

    @functools.partial(
        pl.kernel,
        out_type=jax.ShapeDtypeStruct((rows_total, _DH), jnp.float32),
        mesh=mesh,
        scratch_types=[
            pltpu.VMEM((nchunks, c), jnp.int32),
            pltpu.VMEM((c, _DH), jnp.float32),
            pltpu.VMEM((c, _DH), jnp.float32),
            pltpu.SemaphoreType.DMA,
            pltpu.SemaphoreType.DMA,
        ],
    )
    def gather_kernel(table_hbm, idx_hbm, out_hbm, idx_v, buf0, buf1, sem0, sem1):
        wid = lax.axis_index("s") * 2 + lax.axis_index("c")
        base = wid * per_worker
        pltpu.sync_copy(idx_hbm.at[wid], idx_v)

        bufs = (buf0, buf1)
        sems = (sem0, sem1)
        pltpu.async_copy(table_hbm.at[idx_v.at[0]], buf0, sem0)

        def step(j, _):
            def even_odd(parity):
                buf, sem = bufs[parity], sems[parity]
                nbuf, nsem = bufs[1 - parity], sems[1 - parity]

                @pl.when(j + 1 < nchunks)
                def _():
                    pltpu.async_copy(table_hbm.at[idx_v.at[j + 1]], nbuf, nsem)

                pltpu.make_async_copy(table_hbm.at[idx_v.at[j]], buf, sem).wait()
                pltpu.sync_copy(buf, out_hbm.at[pl.ds(base + j * c, c)])

            @pl.when(j % 2 == 0)
            def _():
                even_odd(0)

            @pl.when(j % 2 == 1)
            def _():
                even_odd(1)

            return 0

        lax.fori_loop(0, nchunks, step, 0)

    return gather_kernel(table, idx)


def _dense(emb, hid, wcat, wg, norms_w, convw):
    n = emb.shape[0]
    t = 256
    grid = n // t
    cpb = 2048 // t
    halo = 16

    def wide(x128):
        return jnp.concatenate([x128] * (_HID // 128), axis=1)

    def body(emb_ref, hid_ref, w_ref, ones_ref, wg_ref, nw_ref, cw_ref,
             out_ref, tail_ref):
        i = pl.program_id(0)
        seq_start = (i % cpb) == 0
        ones = ones_ref[...]

        def rowsum(x):
            return lax.dot_general(x, ones, (((1,), (0,)), ((), ())),
                                   preferred_element_type=jnp.float32)

        eb = emb_ref[...].astype(jnp.bfloat16)
        p = lax.dot_general(eb, w_ref[...],
                            (((1,), (1,)), ((), ())),
                            preferred_element_type=jnp.float32)
        value = p[:, :_HID]
        for m in range(_HC):
            k = p[:, _HID * (m + 1):_HID * (m + 2)]
            q = hid_ref[:, _HID * m:_HID * (m + 1)]
            rsk = lax.rsqrt(rowsum(k * k) * (1.0 / _HID) + _EPS_G)
            rsq = lax.rsqrt(rowsum(q * q) * (1.0 / _HID) + _EPS_G)
            kq = rowsum(k * q * wg_ref[m][None, :])
            g = kq * rsk * rsq * (1.0 / np.sqrt(float(_HID)))
            g = jnp.sqrt(jnp.clip(jnp.abs(g), 1e-6, None)) * jnp.sign(g)
            gate = jax.nn.sigmoid(g)
            gated = wide(gate) * value
            rsg = lax.rsqrt(rowsum(gated * gated) * (1.0 / _HID) + _EPS_C)
            xs = gated * wide(rsg) * nw_ref[m][None, :]
            prev_tail = jnp.where(seq_start, 0.0, tail_ref[m])
            xfull = jnp.concatenate([prev_tail, xs], axis=0)
            co = jnp.zeros_like(xs)
            for kk in range(_KSZ):
                off = halo - (_KSZ - 1) * _DIL + _DIL * kk
                co = co + xfull[off:off + t, :] \
                    * cw_ref[kk, _HID * m:_HID * (m + 1)][None, :]
            co = co * jax.nn.sigmoid(co)
            tail_ref[m] = xs[t - halo:, :]
            out_ref[:, _HID * m:_HID * (m + 1)] = q + co + gated

    return pl.pallas_call(
        body,
        grid=(grid,),
        in_specs=[
            pl.BlockSpec((t, _H * _DH), lambda i: (i, 0)),
            pl.BlockSpec((t, _HC * _HID), lambda i: (i, 0)),
            pl.BlockSpec(((1 + _HC) * _HID, _H * _DH), lambda i: (0, 0)),
            pl.BlockSpec((_HID, 128), lambda i: (0, 0)),
            pl.BlockSpec((_HC, _HID), lambda i: (0, 0)),
            pl.BlockSpec((_HC, _HID), lambda i: (0, 0)),
            pl.BlockSpec((_KSZ, _HC * _HID), lambda i: (0, 0)),
        ],
        out_specs=pl.BlockSpec((t, _HC * _HID), lambda i: (i, 0)),
        out_shape=jax.ShapeDtypeStruct((n, _HC * _HID), jnp.float32),
        scratch_shapes=[pltpu.VMEM((_HC, 16, _HID), jnp.float32)],
    )(emb, hid, wcat, jnp.ones((_HID, 128), jnp.float32), wg, norms_w,
      convw)


def kernel(hash_indices, hidden_states, table, w_v, w_k, g_k, g_h,
           norms_w, conv_w):
    b, s, h = hash_indices.shape
    n = b * s
    idx = hash_indices.astype(jnp.int32) + jnp.asarray(_OFFSETS, jnp.int32)
    idx = idx.reshape(_NW, -1, _CHUNK)
    emb = _sc_gather(table, idx).reshape(n, _H * _DH)
    hid = hidden_states.reshape(n, _HC * _HID)
    wcat = jnp.concatenate(
        [w_v, w_k.reshape(_HC * _HID, _H * _DH)], axis=0
    ).astype(jnp.bfloat16)
    convw = conv_w.reshape(_HC * _HID, _KSZ).T
    out = _dense(emb, hid, wcat, g_k * g_h, norms_w, convw)
    return out.reshape(b, s, _HC, _HID)


# speedup vs baseline: 1.0644x; 1.0043x over previous
import functools

import numpy as np
import jax
import jax.numpy as jnp
from jax import lax
from jax.experimental import pallas as pl
from jax.experimental.pallas import tpu as pltpu
from jax.experimental.pallas import tpu_sc as plsc

_PRIMES = [49999, 49993, 49991, 49957, 49943, 49939, 49937, 49927]
_H = len(_PRIMES)
_HC = 4
_HID = 1024
_DH = 128
_KSZ = 4
_DIL = 3
_EPS_G = float(np.finfo(np.float32).eps)
_EPS_C = 1e-5
_OFFSETS = np.concatenate(
    [[0], np.cumsum(np.asarray(_PRIMES[:-1], dtype=np.int64))]
).astype(np.int32)

_NW = 32
_CHUNK = 128


def _sc_gather(table, idx):
    nw, nchunks, c = idx.shape
    rows_total = nw * nchunks * c
    per_worker = nchunks * c
    mesh = plsc.VectorSubcoreMesh(core_axis_name="c", subcore_axis_name="s")

    @functools.partial(
        pl.kernel,
        out_type=jax.ShapeDtypeStruct((rows_total, _DH), jnp.float32),
        mesh=mesh,
        scratch_types=[
            pltpu.VMEM((nchunks, c), jnp.int32),
            pltpu.VMEM((c, _DH), jnp.float32),
            pltpu.VMEM((c, _DH), jnp.float32),
            pltpu.SemaphoreType.DMA,
            pltpu.SemaphoreType.DMA,
        ],
    )
    def gather_kernel(table_hbm, idx_hbm, out_hbm, idx_v, buf0, buf1, sem0, sem1):
        wid = lax.axis_index("s") * 2 + lax.axis_index("c")
        base = wid * per_worker
        pltpu.sync_copy(idx_hbm.at[wid], idx_v)

        bufs = (buf0, buf1)
        sems = (sem0, sem1)
        pltpu.async_copy(table_hbm.at[idx_v.at[0]], buf0, sem0)

        def step(j, _):
            def even_odd(parity):
                buf, sem = bufs[parity], sems[parity]
                nbuf, nsem = bufs[1 - parity], sems[1 - parity]

                @pl.when(j + 1 < nchunks)
                def _():
                    pltpu.async_copy(table_hbm.at[idx_v.at[j + 1]], nbuf, nsem)

                pltpu.make_async_copy(table_hbm.at[idx_v.at[j]], buf, sem).wait()
                pltpu.sync_copy(buf, out_hbm.at[pl.ds(base + j * c, c)])

            @pl.when(j % 2 == 0)
            def _():
                even_odd(0)

            @pl.when(j % 2 == 1)
            def _():
                even_odd(1)

            return 0

        lax.fori_loop(0, nchunks, step, 0)

    return gather_kernel(table, idx)


def _dense(emb, hid, wcat, wg, norms_w, convw):
    n = emb.shape[0]
    t = 256
    grid = n // t
    cpb = 2048 // t
    halo = 16

    def wide(x128):
        return jnp.concatenate([x128] * (_HID // 128), axis=1)

    def body(emb_ref, hid_ref, w_ref, ones_ref, wg_ref, nw_ref, cw_ref,
             out_ref, tail_ref):
        i = pl.program_id(0)
        seq_start = (i % cpb) == 0
        ones = ones_ref[...]

        def rowsum(x):
            return lax.dot_general(x.astype(jnp.bfloat16), ones,
                                   (((1,), (0,)), ((), ())),
                                   preferred_element_type=jnp.float32)

        eb = emb_ref[...].astype(jnp.bfloat16)
        p = lax.dot_general(eb, w_ref[...],
                            (((1,), (1,)), ((), ())),
                            preferred_element_type=jnp.float32)
        value = p[:, :_HID]
        mv = rowsum(value * value) * (1.0 / _HID)
        for m in range(_HC):
            k = p[:, _HID * (m + 1):_HID * (m + 2)]
            q = hid_ref[:, _HID * m:_HID * (m + 1)]
            rsk = lax.rsqrt(rowsum(k * k) * (1.0 / _HID) + _EPS_G)
            rsq = lax.rsqrt(rowsum(q * q) * (1.0 / _HID) + _EPS_G)
            kq = rowsum(k * q * wg_ref[m][None, :])
            g = kq * rsk * rsq * (1.0 / np.sqrt(float(_HID)))
            g = jnp.sqrt(jnp.clip(jnp.abs(g), 1e-6, None)) * jnp.sign(g)
            gate = jax.nn.sigmoid(g)
            gated = wide(gate) * value
            rsg = lax.rsqrt(gate * gate * mv + _EPS_C)
            xs = gated * wide(rsg) * nw_ref[m][None, :]
            prev_tail = jnp.where(seq_start, 0.0, tail_ref[m])
            xfull = jnp.concatenate([prev_tail, xs], axis=0)
            co = jnp.zeros_like(xs)
            for kk in range(_KSZ):
                off = halo - (_KSZ - 1) * _DIL + _DIL * kk
                co = co + xfull[off:off + t, :] \
                    * cw_ref[kk, _HID * m:_HID * (m + 1)][None, :]
            co = co * jax.nn.sigmoid(co)
            tail_ref[m] = xs[t - halo:, :]
            out_ref[:, _HID * m:_HID * (m + 1)] = q + co + gated

    return pl.pallas_call(
        body,
        grid=(grid,),
        in_specs=[
            pl.BlockSpec((t, _H * _DH), lambda i: (i, 0)),
            pl.BlockSpec((t, _HC * _HID), lambda i: (i, 0)),
            pl.BlockSpec(((1 + _HC) * _HID, _H * _DH), lambda i: (0, 0)),
            pl.BlockSpec((_HID, 128), lambda i: (0, 0)),
            pl.BlockSpec((_HC, _HID), lambda i: (0, 0)),
            pl.BlockSpec((_HC, _HID), lambda i: (0, 0)),
            pl.BlockSpec((_KSZ, _HC * _HID), lambda i: (0, 0)),
        ],
        out_specs=pl.BlockSpec((t, _HC * _HID), lambda i: (i, 0)),
        out_shape=jax.ShapeDtypeStruct((n, _HC * _HID), jnp.float32),
        scratch_shapes=[pltpu.VMEM((_HC, 16, _HID), jnp.float32)],
    )(emb, hid, wcat, jnp.ones((_HID, 128), jnp.bfloat16), wg, norms_w,
      convw)


def kernel(hash_indices, hidden_states, table, w_v, w_k, g_k, g_h,
           norms_w, conv_w):
    b, s, h = hash_indices.shape
    n = b * s
    idx = hash_indices.astype(jnp.int32) + jnp.asarray(_OFFSETS, jnp.int32)
    idx = idx.reshape(_NW, -1, _CHUNK)
    emb = _sc_gather(table, idx).reshape(n, _H * _DH)
    hid = hidden_states.reshape(n, _HC * _HID)
    wcat = jnp.concatenate(
        [w_v, w_k.reshape(_HC * _HID, _H * _DH)], axis=0
    ).astype(jnp.bfloat16)
    convw = conv_w.reshape(_HC * _HID, _KSZ).T
    out = _dense(emb, hid, wcat, g_k * g_h, norms_w, convw)
    return out.reshape(b, s, _HC, _HID)


# speedup vs baseline: 1.1819x; 1.1103x over previous
import functools

import numpy as np
import jax
import jax.numpy as jnp
from jax import lax
from jax.experimental import pallas as pl
from jax.experimental.pallas import tpu as pltpu
from jax.experimental.pallas import tpu_sc as plsc

_PRIMES = [49999, 49993, 49991, 49957, 49943, 49939, 49937, 49927]
_H = len(_PRIMES)
_HC = 4
_HID = 1024
_DH = 128
_KSZ = 4
_DIL = 3
_EPS_G = float(np.finfo(np.float32).eps)
_EPS_C = 1e-5
_OFFSETS = np.concatenate(
    [[0], np.cumsum(np.asarray(_PRIMES[:-1], dtype=np.int64))]
).astype(np.int32)

_NW = 32
_CHUNK = 128


def _sc_gather(table, idx):
    nw, nchunks, c = idx.shape
    rows_total = nw * nchunks * c
    per_worker = nchunks * c
    mesh = plsc.VectorSubcoreMesh(core_axis_name="c", subcore_axis_name="s")

    @functools.partial(
        pl.kernel,
        out_type=jax.ShapeDtypeStruct((rows_total, _DH), jnp.float32),
        mesh=mesh,
        scratch_types=[
            pltpu.VMEM((nchunks, c), jnp.int32),
            pltpu.VMEM((c, _DH), jnp.float32),
            pltpu.VMEM((c, _DH), jnp.float32),
            pltpu.SemaphoreType.DMA,
            pltpu.SemaphoreType.DMA,
        ],
    )
    def gather_kernel(table_hbm, idx_hbm, out_hbm, idx_v, buf0, buf1, sem0, sem1):
        wid = lax.axis_index("s") * 2 + lax.axis_index("c")
        base = wid * per_worker
        pltpu.sync_copy(idx_hbm.at[wid], idx_v)

        bufs = (buf0, buf1)
        sems = (sem0, sem1)
        pltpu.async_copy(table_hbm.at[idx_v.at[0]], buf0, sem0)

        def step(j, _):
            def even_odd(parity):
                buf, sem = bufs[parity], sems[parity]
                nbuf, nsem = bufs[1 - parity], sems[1 - parity]

                @pl.when(j + 1 < nchunks)
                def _():
                    pltpu.async_copy(table_hbm.at[idx_v.at[j + 1]], nbuf, nsem)

                pltpu.make_async_copy(table_hbm.at[idx_v.at[j]], buf, sem).wait()
                pltpu.sync_copy(buf, out_hbm.at[pl.ds(base + j * c, c)])

            @pl.when(j % 2 == 0)
            def _():
                even_odd(0)

            @pl.when(j % 2 == 1)
            def _():
                even_odd(1)

            return 0

        lax.fori_loop(0, nchunks, step, 0)

    return gather_kernel(table, idx)


def _dense(emb, hid, wcat, wg, norms_w, convw):
    n = emb.shape[0]
    t = 256
    grid = n // t
    cpb = 2048 // t
    halo = 16

    def body(emb_ref, hid_ref, w_ref, pmat_ref, wg_ref, nw_ref, cw_ref,
             out_ref, tail_ref):
        i = pl.program_id(0)
        seq_start = (i % cpb) == 0
        eb = emb_ref[...].astype(jnp.bfloat16)
        p = lax.dot_general(eb, w_ref[...],
                            (((1,), (1,)), ((), ())),
                            preferred_element_type=jnp.float32)
        value = p[:, :_HID]
        mv = jnp.mean(value * value, axis=-1, keepdims=True)
        for m in range(_HC):
            k = p[:, _HID * (m + 1):_HID * (m + 2)]
            q = hid_ref[:, _HID * m:_HID * (m + 1)]
            rsk = lax.rsqrt(jnp.mean(k * k, axis=-1, keepdims=True) + _EPS_G)
            rsq = lax.rsqrt(jnp.mean(q * q, axis=-1, keepdims=True) + _EPS_G)
            kq = jnp.sum(k * q * wg_ref[m][None, :], axis=-1, keepdims=True)
            g = kq * rsk * rsq * (1.0 / np.sqrt(float(_HID)))
            g = jnp.sqrt(jnp.clip(jnp.abs(g), 1e-6, None)) * jnp.sign(g)
            gate = jax.nn.sigmoid(g)
            gated = gate * value
            rsg = lax.rsqrt(gate * gate * mv + _EPS_C)
            xs = gated * rsg * nw_ref[m][None, :]
            prev_tail = jnp.where(seq_start, 0.0, tail_ref[m])
            xfull = jnp.concatenate(
                [prev_tail, xs], axis=0).astype(jnp.bfloat16)
            sh = lax.dot_general(pmat_ref[...], xfull,
                                 (((1,), (0,)), ((), ())),
                                 preferred_element_type=jnp.float32)
            co = jnp.zeros_like(xs)
            for kk in range(_KSZ):
                co = co + sh[t * kk:t * (kk + 1), :] \
                    * cw_ref[kk, _HID * m:_HID * (m + 1)][None, :]
            co = co * jax.nn.sigmoid(co)
            tail_ref[m] = xs[t - halo:, :]
            out_ref[:, _HID * m:_HID * (m + 1)] = q + co + gated

    call = pl.pallas_call(
        body,
        grid=(grid,),
        in_specs=[
            pl.BlockSpec((t, _H * _DH), lambda i: (i, 0)),
            pl.BlockSpec((t, _HC * _HID), lambda i: (i, 0)),
            pl.BlockSpec(((1 + _HC) * _HID, _H * _DH), lambda i: (0, 0)),
            pl.BlockSpec((_KSZ * t, t + halo), lambda i: (0, 0)),
            pl.BlockSpec((_HC, _HID), lambda i: (0, 0)),
            pl.BlockSpec((_HC, _HID), lambda i: (0, 0)),
            pl.BlockSpec((_KSZ, _HC * _HID), lambda i: (0, 0)),
        ],
        out_specs=pl.BlockSpec((t, _HC * _HID), lambda i: (i, 0)),
        out_shape=jax.ShapeDtypeStruct((n, _HC * _HID), jnp.float32),
        scratch_shapes=[pltpu.VMEM((_HC, 16, _HID), jnp.float32)],
    )
    pmat = np.zeros((_KSZ * t, t + halo), np.float32)
    for kk in range(_KSZ):
        off = halo - (_KSZ - 1) * _DIL + _DIL * kk
        pmat[t * kk + np.arange(t), np.arange(t) + off] = 1.0
    return call(emb, hid, wcat, jnp.asarray(pmat, jnp.bfloat16), wg,
                norms_w, convw)


def kernel(hash_indices, hidden_states, table, w_v, w_k, g_k, g_h,
           norms_w, conv_w):
    b, s, h = hash_indices.shape
    n = b * s
    idx = hash_indices.astype(jnp.int32) + jnp.asarray(_OFFSETS, jnp.int32)
    idx = idx.reshape(_NW, -1, _CHUNK)
    emb = _sc_gather(table, idx).reshape(n, _H * _DH)
    hid = hidden_states.reshape(n, _HC * _HID)
    wcat = jnp.concatenate(
        [w_v, w_k.reshape(_HC * _HID, _H * _DH)], axis=0
    ).astype(jnp.bfloat16)
    convw = conv_w.reshape(_HC * _HID, _KSZ).T
    out = _dense(emb, hid, wcat, g_k * g_h, norms_w, convw)
    return out.reshape(b, s, _HC, _HID)


# speedup vs baseline: 1.2259x; 1.0373x over previous
import functools

import numpy as np
import jax
import jax.numpy as jnp
from jax import lax
from jax.experimental import pallas as pl
from jax.experimental.pallas import tpu as pltpu
from jax.experimental.pallas import tpu_sc as plsc

_PRIMES = [49999, 49993, 49991, 49957, 49943, 49939, 49937, 49927]
_H = len(_PRIMES)
_HC = 4
_HID = 1024
_DH = 128
_KSZ = 4
_DIL = 3
_EPS_G = float(np.finfo(np.float32).eps)
_EPS_C = 1e-5
_OFFSETS = np.concatenate(
    [[0], np.cumsum(np.asarray(_PRIMES[:-1], dtype=np.int64))]
).astype(np.int32)

_NW = 32
_CHUNK = 128


def _sc_gather(table, idx, n_tokens):
    nw, nchunks, c = idx.shape
    mesh = plsc.VectorSubcoreMesh(core_axis_name="c", subcore_axis_name="s")

    @functools.partial(
        pl.kernel,
        out_type=jax.ShapeDtypeStruct((n_tokens, _H * _DH), jnp.float32),
        mesh=mesh,
        scratch_types=[
            pltpu.VMEM((nchunks, c), jnp.int32),
            pltpu.VMEM((c, _DH), jnp.float32),
            pltpu.VMEM((c, _DH), jnp.float32),
            pltpu.SemaphoreType.DMA,
            pltpu.SemaphoreType.DMA,
        ],
    )
    def gather_kernel(table_hbm, idx_hbm, out_hbm, idx_v, buf0, buf1, sem0, sem1):
        wid = lax.axis_index("s") * 2 + lax.axis_index("c")
        pltpu.sync_copy(idx_hbm.at[wid], idx_v)

        bufs = (buf0, buf1)
        sems = (sem0, sem1)
        pltpu.async_copy(table_hbm.at[idx_v.at[0]], buf0, sem0)

        def step(j, _):
            g = wid * nchunks + j
            tb = g // _H
            h = g % _H

            def even_odd(parity):
                buf, sem = bufs[parity], sems[parity]
                nbuf, nsem = bufs[1 - parity], sems[1 - parity]

                @pl.when(j + 1 < nchunks)
                def _():
                    pltpu.async_copy(table_hbm.at[idx_v.at[j + 1]], nbuf, nsem)

                pltpu.make_async_copy(table_hbm.at[idx_v.at[j]], buf, sem).wait()
                pltpu.sync_copy(
                    buf,
                    out_hbm.at[pl.ds(tb * c, c), pl.ds(h * _DH, _DH)],
                )

            @pl.when(j % 2 == 0)
            def _():
                even_odd(0)

            @pl.when(j % 2 == 1)
            def _():
                even_odd(1)

            return 0

        lax.fori_loop(0, nchunks, step, 0)

    return gather_kernel(table, idx)


def _dense(emb, hid, wcat, wg, norms_w, convw):
    n = emb.shape[0]
    t = 256
    grid = n // t
    cpb = 2048 // t
    halo = 16

    def body(emb_ref, hid_ref, w_ref, pmat_ref, wg_ref, nw_ref, cw_ref,
             out_ref, tail_ref):
        i = pl.program_id(0)
        seq_start = (i % cpb) == 0
        eb = emb_ref[...].astype(jnp.bfloat16)
        p = lax.dot_general(eb, w_ref[...],
                            (((1,), (1,)), ((), ())),
                            preferred_element_type=jnp.float32)
        value = p[:, :_HID]
        mv = jnp.mean(value * value, axis=-1, keepdims=True)
        for m in range(_HC):
            k = p[:, _HID * (m + 1):_HID * (m + 2)]
            q = hid_ref[:, _HID * m:_HID * (m + 1)]
            rsk = lax.rsqrt(jnp.mean(k * k, axis=-1, keepdims=True) + _EPS_G)
            rsq = lax.rsqrt(jnp.mean(q * q, axis=-1, keepdims=True) + _EPS_G)
            kq = jnp.sum(k * q * wg_ref[m][None, :], axis=-1, keepdims=True)
            g = kq * rsk * rsq * (1.0 / np.sqrt(float(_HID)))
            g = jnp.sqrt(jnp.clip(jnp.abs(g), 1e-6, None)) * jnp.sign(g)
            gate = jax.nn.sigmoid(g)
            gated = gate * value
            rsg = lax.rsqrt(gate * gate * mv + _EPS_C)
            xs = gated * rsg * nw_ref[m][None, :]
            prev_tail = jnp.where(seq_start, 0.0, tail_ref[m])
            xfull = jnp.concatenate(
                [prev_tail, xs], axis=0).astype(jnp.bfloat16)
            sh = lax.dot_general(pmat_ref[...], xfull,
                                 (((1,), (0,)), ((), ())),
                                 preferred_element_type=jnp.float32)
            co = jnp.zeros_like(xs)
            for kk in range(_KSZ):
                co = co + sh[t * kk:t * (kk + 1), :] \
                    * cw_ref[kk, _HID * m:_HID * (m + 1)][None, :]
            co = co * jax.nn.sigmoid(co)
            tail_ref[m] = xs[t - halo:, :]
            out_ref[:, _HID * m:_HID * (m + 1)] = q + co + gated

    call = pl.pallas_call(
        body,
        grid=(grid,),
        in_specs=[
            pl.BlockSpec((t, _H * _DH), lambda i: (i, 0)),
            pl.BlockSpec((t, _HC * _HID), lambda i: (i, 0)),
            pl.BlockSpec(((1 + _HC) * _HID, _H * _DH), lambda i: (0, 0)),
            pl.BlockSpec((_KSZ * t, t + halo), lambda i: (0, 0)),
            pl.BlockSpec((_HC, _HID), lambda i: (0, 0)),
            pl.BlockSpec((_HC, _HID), lambda i: (0, 0)),
            pl.BlockSpec((_KSZ, _HC * _HID), lambda i: (0, 0)),
        ],
        out_specs=pl.BlockSpec((t, _HC * _HID), lambda i: (i, 0)),
        out_shape=jax.ShapeDtypeStruct((n, _HC * _HID), jnp.float32),
        scratch_shapes=[pltpu.VMEM((_HC, 16, _HID), jnp.float32)],
    )
    pmat = np.zeros((_KSZ * t, t + halo), np.float32)
    for kk in range(_KSZ):
        off = halo - (_KSZ - 1) * _DIL + _DIL * kk
        pmat[t * kk + np.arange(t), np.arange(t) + off] = 1.0
    return call(emb, hid, wcat, jnp.asarray(pmat, jnp.bfloat16), wg,
                norms_w, convw)


def kernel(hash_indices, hidden_states, table, w_v, w_k, g_k, g_h,
           norms_w, conv_w):
    b, s, h = hash_indices.shape
    n = b * s
    idx = hash_indices.astype(jnp.int32) + jnp.asarray(_OFFSETS, jnp.int32)
    idx = (idx.reshape(n // _CHUNK, _CHUNK, _H)
           .transpose(0, 2, 1)
           .reshape(_NW, -1, _CHUNK))
    emb = _sc_gather(table, idx, n)
    hid = hidden_states.reshape(n, _HC * _HID)
    wcat = jnp.concatenate(
        [w_v, w_k.reshape(_HC * _HID, _H * _DH)], axis=0
    ).astype(jnp.bfloat16)
    convw = conv_w.reshape(_HC * _HID, _KSZ).T
    out = _dense(emb, hid, wcat, g_k * g_h, norms_w, convw)
    return out.reshape(b, s, _HC, _HID)


# speedup vs baseline: 1.6591x; 1.3534x over previous
import functools

import numpy as np
import jax
import jax.numpy as jnp
from jax import lax
from jax.experimental import pallas as pl
from jax.experimental.pallas import tpu as pltpu
from jax.experimental.pallas import tpu_sc as plsc

_PRIMES = [49999, 49993, 49991, 49957, 49943, 49939, 49937, 49927]
_H = len(_PRIMES)
_HC = 4
_HID = 1024
_DH = 128
_KSZ = 4
_DIL = 3
_EPS_G = float(np.finfo(np.float32).eps)
_EPS_C = 1e-5
_OFFSETS = np.concatenate(
    [[0], np.cumsum(np.asarray(_PRIMES[:-1], dtype=np.int64))]
).astype(np.int32)

_NW = 32
_CHUNK = 128


def _sc_gather(table, idx, n_tokens):
    nw, nchunks, c = idx.shape
    mesh = plsc.VectorSubcoreMesh(core_axis_name="c", subcore_axis_name="s")

    @functools.partial(
        pl.kernel,
        out_type=jax.ShapeDtypeStruct((n_tokens, _H * _DH), jnp.float32),
        mesh=mesh,
        scratch_types=[
            pltpu.VMEM((nchunks, c), jnp.int32),
            pltpu.VMEM((c, _DH), jnp.float32),
            pltpu.VMEM((c, _DH), jnp.float32),
            pltpu.SemaphoreType.DMA,
            pltpu.SemaphoreType.DMA,
        ],
    )
    def gather_kernel(table_hbm, idx_hbm, out_hbm, idx_v, buf0, buf1, sem0, sem1):
        wid = lax.axis_index("s") * 2 + lax.axis_index("c")
        pltpu.sync_copy(idx_hbm.at[wid], idx_v)

        bufs = (buf0, buf1)
        sems = (sem0, sem1)
        pltpu.async_copy(table_hbm.at[idx_v.at[0]], buf0, sem0)

        def step(j, _):
            g = wid * nchunks + j
            tb = g // _H
            h = g % _H

            def even_odd(parity):
                buf, sem = bufs[parity], sems[parity]
                nbuf, nsem = bufs[1 - parity], sems[1 - parity]

                @pl.when(j + 1 < nchunks)
                def _():
                    pltpu.async_copy(table_hbm.at[idx_v.at[j + 1]], nbuf, nsem)

                pltpu.make_async_copy(table_hbm.at[idx_v.at[j]], buf, sem).wait()
                pltpu.sync_copy(
                    buf,
                    out_hbm.at[pl.ds(tb * c, c), pl.ds(h * _DH, _DH)],
                )

            @pl.when(j % 2 == 0)
            def _():
                even_odd(0)

            @pl.when(j % 2 == 1)
            def _():
                even_odd(1)

            return 0

        lax.fori_loop(0, nchunks, step, 0)

    return gather_kernel(table, idx)


def _dense(emb, hid, wcat, wg, norms_w, convw):
    n = emb.shape[0]
    t = 256
    grid = n // t
    cpb = 2048 // t
    halo = 16

    def body(emb_ref, hid_ref, w_ref, pmat_ref, wg_ref, nw_ref, cw_ref,
             out_ref, tail_ref):
        i = pl.program_id(0)
        seq_start = (i % cpb) == 0
        eb = emb_ref[...].astype(jnp.bfloat16)
        p = lax.dot_general(eb, w_ref[...],
                            (((1,), (1,)), ((), ())),
                            preferred_element_type=jnp.float32)
        value = p[:, :_HID]
        mv = jnp.mean(value * value, axis=-1, keepdims=True)
        for m in range(_HC):
            k = p[:, _HID * (m + 1):_HID * (m + 2)]
            q = hid_ref[:, m, :]
            rsk = lax.rsqrt(jnp.mean(k * k, axis=-1, keepdims=True) + _EPS_G)
            rsq = lax.rsqrt(jnp.mean(q * q, axis=-1, keepdims=True) + _EPS_G)
            kq = jnp.sum(k * q * wg_ref[m][None, :], axis=-1, keepdims=True)
            g = kq * rsk * rsq * (1.0 / np.sqrt(float(_HID)))
            g = jnp.sqrt(jnp.clip(jnp.abs(g), 1e-6, None)) * jnp.sign(g)
            gate = jax.nn.sigmoid(g)
            gated = gate * value
            rsg = lax.rsqrt(gate * gate * mv + _EPS_C)
            xs = gated * rsg * nw_ref[m][None, :]
            prev_tail = jnp.where(seq_start, 0.0, tail_ref[m])
            xfull = jnp.concatenate(
                [prev_tail, xs], axis=0).astype(jnp.bfloat16)
            sh = lax.dot_general(pmat_ref[...], xfull,
                                 (((1,), (0,)), ((), ())),
                                 preferred_element_type=jnp.float32)
            co = jnp.zeros_like(xs)
            for kk in range(_KSZ):
                co = co + sh[t * kk:t * (kk + 1), :] \
                    * cw_ref[kk, _HID * m:_HID * (m + 1)][None, :]
            co = co * jax.nn.sigmoid(co)
            tail_ref[m] = xs[t - halo:, :]
            out_ref[:, m, :] = q + co + gated

    call = pl.pallas_call(
        body,
        grid=(grid,),
        in_specs=[
            pl.BlockSpec((t, _H * _DH), lambda i: (i, 0)),
            pl.BlockSpec((t, _HC, _HID), lambda i: (i, 0, 0)),
            pl.BlockSpec(((1 + _HC) * _HID, _H * _DH), lambda i: (0, 0)),
            pl.BlockSpec((_KSZ * t, t + halo), lambda i: (0, 0)),
            pl.BlockSpec((_HC, _HID), lambda i: (0, 0)),
            pl.BlockSpec((_HC, _HID), lambda i: (0, 0)),
            pl.BlockSpec((_KSZ, _HC * _HID), lambda i: (0, 0)),
        ],
        out_specs=pl.BlockSpec((t, _HC, _HID), lambda i: (i, 0, 0)),
        out_shape=jax.ShapeDtypeStruct((n, _HC, _HID), jnp.float32),
        scratch_shapes=[pltpu.VMEM((_HC, 16, _HID), jnp.float32)],
    )
    pmat = np.zeros((_KSZ * t, t + halo), np.float32)
    for kk in range(_KSZ):
        off = halo - (_KSZ - 1) * _DIL + _DIL * kk
        pmat[t * kk + np.arange(t), np.arange(t) + off] = 1.0
    return call(emb, hid, wcat, jnp.asarray(pmat, jnp.bfloat16), wg,
                norms_w, convw)


def kernel(hash_indices, hidden_states, table, w_v, w_k, g_k, g_h,
           norms_w, conv_w):
    b, s, h = hash_indices.shape
    n = b * s
    idx = hash_indices.astype(jnp.int32) + jnp.asarray(_OFFSETS, jnp.int32)
    idx = (idx.reshape(n // _CHUNK, _CHUNK, _H)
           .transpose(0, 2, 1)
           .reshape(_NW, -1, _CHUNK))
    emb = _sc_gather(table, idx, n)
    hid = hidden_states.reshape(n, _HC, _HID)
    wcat = jnp.concatenate(
        [w_v, w_k.reshape(_HC * _HID, _H * _DH)], axis=0
    ).astype(jnp.bfloat16)
    convw = conv_w.reshape(_HC * _HID, _KSZ).T
    out = _dense(emb, hid, wcat, g_k * g_h, norms_w, convw)
    return out.reshape(b, s, _HC, _HID)
